# Initial kernel scaffold; baseline (speedup 1.0000x reference)
import jax
import jax.numpy as jnp
from jax.experimental import pallas as pl


def kernel(edge_index, x, W_self1, W_neigh1, b1, W_self2, W_neigh2, b2):
    raise NotImplementedError("write your pallas kernel here")


# speedup vs baseline: 3.2416x; 3.2416x over previous
import functools

import jax
import jax.numpy as jnp
from jax import lax
from jax.experimental import pallas as pl
from jax.experimental.pallas import tpu as pltpu
from jax.experimental.pallas import tpu_sc as plsc

N = 10000
E = 160000
D = 256
H = 128
NP = 10240
EP = 163840
NS = 16
NCH = EP // NS // 128
RPT = NP // NS
TRASH = N

_f32 = jnp.float32


def _sc_agg():
    mesh = plsc.VectorSubcoreMesh(core_axis_name="c", subcore_axis_name="s",
                                  num_cores=2, num_subcores=NS)
    out_type = [
        jax.ShapeDtypeStruct((NP, H), _f32),
        jax.ShapeDtypeStruct((NP, H), _f32),
    ]
    scratch = [
        pltpu.VMEM((NCH, 128), jnp.int32),
        pltpu.VMEM((NCH, 128), jnp.int32),
        pltpu.VMEM((128, H), _f32),
        pltpu.VMEM_SHARED((NP, H), _f32),
        pltpu.SemaphoreType.DMA,
    ]

    def body(src_hbm, dst_hbm, m0_hbm, m1_hbm, *refs):
        (out0, out1, src_v, dst_v, rows, acc, sem) = refs
        c = lax.axis_index("c")
        s = lax.axis_index("s")

        pltpu.sync_copy(src_hbm.at[s], src_v)
        pltpu.sync_copy(dst_hbm.at[s], dst_v)

        z16 = jnp.zeros((16,), _f32)

        @pl.loop(0, 128)
        def _(r):
            for k in range(H // 16):
                rows[r, pl.ds(k * 16, 16)] = z16

        for r in range(RPT // 128):
            pltpu.sync_copy(rows, acc.at[pl.ds(s * RPT + r * 128, 128)])

        plsc.subcore_barrier()

        def run(m_hbm):
            @pl.loop(0, NCH)
            def _(j):
                pltpu.async_copy(m_hbm.at[src_v.at[j]], rows, sem).wait()
                pltpu.sync_copy(rows, acc.at[dst_v.at[j]], add=True)

        @pl.when(c == 0)
        def _():
            run(m0_hbm)

        @pl.when(c == 1)
        def _():
            run(m1_hbm)

        plsc.subcore_barrier()

        sl = pl.ds(s * RPT, RPT)

        @pl.when(c == 0)
        def _():
            pltpu.sync_copy(acc.at[sl], out0.at[sl])

        @pl.when(c == 1)
        def _():
            pltpu.sync_copy(acc.at[sl], out1.at[sl])

    return pl.kernel(body, out_type=out_type, mesh=mesh, scratch_types=scratch)


def _tc_layer(relu, split):
    BM = 1000
    grid = (N // BM,)
    in_specs = [
        pl.BlockSpec((BM, H), lambda i: (i, 0)),
        pl.BlockSpec((BM, H), lambda i: (i, 0)),
        pl.BlockSpec((BM, H), lambda i: (i, 0)),
        pl.BlockSpec((BM, H), lambda i: (i, 0)),
        pl.BlockSpec((BM, H), lambda i: (i, 0)),
        pl.BlockSpec((BM, H), lambda i: (i, 0)),
        pl.BlockSpec((2 * D, D), lambda i: (0, 0)),
        pl.BlockSpec((1, D), lambda i: (0, 0)),
    ]
    if split:
        out_specs = [
            pl.BlockSpec((BM, H), lambda i: (i, 0)),
            pl.BlockSpec((BM, H), lambda i: (i, 0)),
        ]
        out_shape = [
            jax.ShapeDtypeStruct((N, H), _f32),
            jax.ShapeDtypeStruct((N, H), _f32),
        ]
    else:
        out_specs = pl.BlockSpec((BM, D), lambda i: (i, 0))
        out_shape = jax.ShapeDtypeStruct((N, D), _f32)

    def body(x0, x1, a0, a1, d0, d1, v, b, *outs):
        deg = d0[...][:, :1] + d1[...][:, :1]
        rec = 1.0 / jnp.maximum(deg, 1.0)
        xin = jnp.concatenate(
            [x0[...], x1[...], a0[...] * rec, a1[...] * rec], axis=1)
        acc = jnp.dot(xin, v[...], preferred_element_type=_f32) + b[...]
        if relu:
            acc = jnp.maximum(acc, 0.0)
        if split:
            outs[0][...] = acc[:, :H]
            outs[1][...] = acc[:, H:]
        else:
            outs[0][...] = acc

    return pl.pallas_call(body, grid=grid, in_specs=in_specs,
                          out_specs=out_specs, out_shape=out_shape)


NCH2 = EP // (2 * NS) // 128


def _sc_deg():
    mesh = plsc.VectorSubcoreMesh(core_axis_name="c", subcore_axis_name="s",
                                  num_cores=2, num_subcores=NS)
    out_type = [
        jax.ShapeDtypeStruct((NP, H), _f32),
        jax.ShapeDtypeStruct((NP, H), _f32),
    ]
    scratch = [
        pltpu.VMEM((NCH2, 128), jnp.int32),
        pltpu.VMEM((128, H), _f32),
        pltpu.VMEM_SHARED((NP, H), _f32),
    ]

    def body(dst_hbm, out0, out1, dst_v, ones_v, degacc):
        c = lax.axis_index("c")
        s = lax.axis_index("s")
        wid = c * NS + s

        pltpu.sync_copy(dst_hbm.at[wid], dst_v)

        z16 = jnp.zeros((16,), _f32)
        o16 = jnp.ones((16,), _f32)

        @pl.loop(0, 128)
        def _(r):
            for k in range(H // 16):
                ones_v[r, pl.ds(k * 16, 16)] = z16

        for r in range(RPT // 128):
            pltpu.sync_copy(ones_v, degacc.at[pl.ds(s * RPT + r * 128, 128)])

        @pl.loop(0, 128)
        def _(r):
            for k in range(H // 16):
                ones_v[r, pl.ds(k * 16, 16)] = o16

        plsc.subcore_barrier()

        @pl.loop(0, NCH2)
        def _(j):
            pltpu.sync_copy(ones_v, degacc.at[dst_v.at[j]], add=True)

        plsc.subcore_barrier()

        sl = pl.ds(s * RPT, RPT)

        @pl.when(c == 0)
        def _():
            pltpu.sync_copy(degacc.at[sl], out0.at[sl])

        @pl.when(c == 1)
        def _():
            pltpu.sync_copy(degacc.at[sl], out1.at[sl])

    return pl.kernel(body, out_type=out_type, mesh=mesh, scratch_types=scratch)


_sc_deg_k = _sc_deg()
_sc_agg_k = _sc_agg()
_tc1 = _tc_layer(relu=True, split=True)
_tc2 = _tc_layer(relu=False, split=False)


@jax.jit
def kernel(edge_index, x, W_self1, W_neigh1, b1, W_self2, W_neigh2, b2):
    src = edge_index[0].astype(jnp.int32)
    dst = edge_index[1].astype(jnp.int32)
    pad = EP - E
    src_r = jnp.concatenate([src, jnp.zeros((pad,), jnp.int32)]).reshape(NS, NCH, 128)
    dst_r = jnp.concatenate([dst, jnp.full((pad,), TRASH, jnp.int32)]).reshape(NS, NCH, 128)

    x0 = x[:, :H]
    x1 = x[:, H:]
    V1 = jnp.concatenate([W_self1.T, W_neigh1.T], axis=0)
    V2 = jnp.concatenate([W_self2.T, W_neigh2.T], axis=0)
    b1r = b1.reshape(1, D)
    b2r = b2.reshape(1, D)

    dst_r2 = dst_r.reshape(2 * NS, NCH2, 128)
    deg0, deg1 = _sc_deg_k(dst_r2)
    agg0, agg1 = _sc_agg_k(src_r, dst_r, x0, x1)
    h0, h1 = _tc1(x0, x1, agg0, agg1, deg0, deg1, V1, b1r)
    ah0, ah1 = _sc_agg_k(src_r, dst_r, h0, h1)
    return _tc2(h0, h1, ah0, ah1, deg0, deg1, V2, b2r)


# speedup vs baseline: 3.8468x; 1.1867x over previous
import functools

import jax
import jax.numpy as jnp
from jax import lax
from jax.experimental import pallas as pl
from jax.experimental.pallas import tpu as pltpu
from jax.experimental.pallas import tpu_sc as plsc

N = 10000
E = 160000
D = 256
H = 128
NP = 10240
EP = 163840
NS = 16
NCH = EP // NS // 128
RPT = NP // NS
TRASH = N

_f32 = jnp.float32


def _sc_agg():
    mesh = plsc.VectorSubcoreMesh(core_axis_name="c", subcore_axis_name="s",
                                  num_cores=2, num_subcores=NS)
    out_type = [
        jax.ShapeDtypeStruct((NP, H), _f32),
        jax.ShapeDtypeStruct((NP, H), _f32),
    ]
    HC = NCH // 2
    scratch = [
        pltpu.VMEM((HC, 128), jnp.int32),
        pltpu.VMEM((HC, 128), jnp.int32),
        pltpu.VMEM((128, H), _f32),
        pltpu.VMEM((128, H), _f32),
        pltpu.VMEM_SHARED((NP, H), _f32),
        pltpu.SemaphoreType.DMA,
        pltpu.SemaphoreType.DMA,
        pltpu.SemaphoreType.DMA,
        pltpu.SemaphoreType.DMA,
    ]

    def body(src_hbm, dst_hbm, m0_hbm, m1_hbm, *refs):
        (out0, out1, src_v, dst_v, rows0, rows1, acc, g0, g1, s0, s1) = refs
        rows = (rows0, rows1)
        gsem = (g0, g1)
        ssem = (s0, s1)
        c = lax.axis_index("c")
        s = lax.axis_index("s")

        z16 = jnp.zeros((16,), _f32)

        @pl.loop(0, 128)
        def _(r):
            for k in range(H // 16):
                rows0[r, pl.ds(k * 16, 16)] = z16

        for r in range(RPT // 128):
            pltpu.sync_copy(rows0, acc.at[pl.ds(s * RPT + r * 128, 128)])

        plsc.subcore_barrier()

        def run(m_hbm):
            for half in range(2):
                pltpu.sync_copy(src_hbm.at[s, pl.ds(half * HC, HC)], src_v)
                pltpu.sync_copy(dst_hbm.at[s, pl.ds(half * HC, HC)], dst_v)
                pltpu.async_copy(m_hbm.at[src_v.at[0]], rows0, g0)
                pltpu.async_copy(m_hbm.at[src_v.at[1]], rows1, g1)

                @pl.loop(0, HC, step=2)
                def _(jj):
                    for b in range(2):
                        j = jj + b
                        pltpu.make_async_copy(
                            m_hbm.at[src_v.at[j]], rows[b], gsem[b]).wait()
                        pltpu.async_copy(
                            rows[b], acc.at[dst_v.at[j]], ssem[b], add=True)

                        @pl.when(j + 2 < HC)
                        def _():
                            pltpu.make_async_copy(
                                rows[b], acc.at[dst_v.at[j]], ssem[b]).wait()
                            pltpu.async_copy(
                                m_hbm.at[src_v.at[j + 2]], rows[b], gsem[b])

                for b in range(2):
                    pltpu.make_async_copy(
                        rows[b], acc.at[dst_v.at[HC - 2 + b]], ssem[b]).wait()

        @pl.when(c == 0)
        def _():
            run(m0_hbm)

        @pl.when(c == 1)
        def _():
            run(m1_hbm)

        plsc.subcore_barrier()

        sl = pl.ds(s * RPT, RPT)

        @pl.when(c == 0)
        def _():
            pltpu.sync_copy(acc.at[sl], out0.at[sl])

        @pl.when(c == 1)
        def _():
            pltpu.sync_copy(acc.at[sl], out1.at[sl])

    return pl.kernel(body, out_type=out_type, mesh=mesh, scratch_types=scratch)


def _tc_layer(relu, split):
    BM = 1000
    grid = (N // BM,)
    in_specs = [
        pl.BlockSpec((BM, H), lambda i: (i, 0)),
        pl.BlockSpec((BM, H), lambda i: (i, 0)),
        pl.BlockSpec((BM, H), lambda i: (i, 0)),
        pl.BlockSpec((BM, H), lambda i: (i, 0)),
        pl.BlockSpec((BM, H), lambda i: (i, 0)),
        pl.BlockSpec((BM, H), lambda i: (i, 0)),
        pl.BlockSpec((2 * D, D), lambda i: (0, 0)),
        pl.BlockSpec((1, D), lambda i: (0, 0)),
    ]
    if split:
        out_specs = [
            pl.BlockSpec((BM, H), lambda i: (i, 0)),
            pl.BlockSpec((BM, H), lambda i: (i, 0)),
        ]
        out_shape = [
            jax.ShapeDtypeStruct((N, H), _f32),
            jax.ShapeDtypeStruct((N, H), _f32),
        ]
    else:
        out_specs = pl.BlockSpec((BM, D), lambda i: (i, 0))
        out_shape = jax.ShapeDtypeStruct((N, D), _f32)

    def body(x0, x1, a0, a1, d0, d1, v, b, *outs):
        deg = d0[...][:, :1] + d1[...][:, :1]
        rec = 1.0 / jnp.maximum(deg, 1.0)
        xin = jnp.concatenate(
            [x0[...], x1[...], a0[...] * rec, a1[...] * rec], axis=1)
        acc = jnp.dot(xin, v[...], preferred_element_type=_f32) + b[...]
        if relu:
            acc = jnp.maximum(acc, 0.0)
        if split:
            outs[0][...] = acc[:, :H]
            outs[1][...] = acc[:, H:]
        else:
            outs[0][...] = acc

    return pl.pallas_call(body, grid=grid, in_specs=in_specs,
                          out_specs=out_specs, out_shape=out_shape)


NCH2 = EP // (2 * NS) // 128


def _sc_deg():
    mesh = plsc.VectorSubcoreMesh(core_axis_name="c", subcore_axis_name="s",
                                  num_cores=2, num_subcores=NS)
    out_type = [
        jax.ShapeDtypeStruct((NP, H), _f32),
        jax.ShapeDtypeStruct((NP, H), _f32),
    ]
    scratch = [
        pltpu.VMEM((NCH2, 128), jnp.int32),
        pltpu.VMEM((128, H), _f32),
        pltpu.VMEM_SHARED((NP, H), _f32),
    ]

    def body(dst_hbm, out0, out1, dst_v, ones_v, degacc):
        c = lax.axis_index("c")
        s = lax.axis_index("s")
        wid = c * NS + s

        pltpu.sync_copy(dst_hbm.at[wid], dst_v)

        z16 = jnp.zeros((16,), _f32)
        o16 = jnp.ones((16,), _f32)

        @pl.loop(0, 128)
        def _(r):
            for k in range(H // 16):
                ones_v[r, pl.ds(k * 16, 16)] = z16

        for r in range(RPT // 128):
            pltpu.sync_copy(ones_v, degacc.at[pl.ds(s * RPT + r * 128, 128)])

        @pl.loop(0, 128)
        def _(r):
            for k in range(H // 16):
                ones_v[r, pl.ds(k * 16, 16)] = o16

        plsc.subcore_barrier()

        @pl.loop(0, NCH2)
        def _(j):
            pltpu.sync_copy(ones_v, degacc.at[dst_v.at[j]], add=True)

        plsc.subcore_barrier()

        sl = pl.ds(s * RPT, RPT)

        @pl.when(c == 0)
        def _():
            pltpu.sync_copy(degacc.at[sl], out0.at[sl])

        @pl.when(c == 1)
        def _():
            pltpu.sync_copy(degacc.at[sl], out1.at[sl])

    return pl.kernel(body, out_type=out_type, mesh=mesh, scratch_types=scratch)


_sc_deg_k = _sc_deg()
_sc_agg_k = _sc_agg()
_tc1 = _tc_layer(relu=True, split=True)
_tc2 = _tc_layer(relu=False, split=False)


@jax.jit
def kernel(edge_index, x, W_self1, W_neigh1, b1, W_self2, W_neigh2, b2):
    src = edge_index[0].astype(jnp.int32)
    dst = edge_index[1].astype(jnp.int32)
    pad = EP - E
    src_r = jnp.concatenate([src, jnp.zeros((pad,), jnp.int32)]).reshape(NS, NCH, 128)
    dst_r = jnp.concatenate([dst, jnp.full((pad,), TRASH, jnp.int32)]).reshape(NS, NCH, 128)

    x0 = x[:, :H]
    x1 = x[:, H:]
    V1 = jnp.concatenate([W_self1.T, W_neigh1.T], axis=0)
    V2 = jnp.concatenate([W_self2.T, W_neigh2.T], axis=0)
    b1r = b1.reshape(1, D)
    b2r = b2.reshape(1, D)

    dst_r2 = dst_r.reshape(2 * NS, NCH2, 128)
    deg0, deg1 = _sc_deg_k(dst_r2)
    agg0, agg1 = _sc_agg_k(src_r, dst_r, x0, x1)
    h0, h1 = _tc1(x0, x1, agg0, agg1, deg0, deg1, V1, b1r)
    ah0, ah1 = _sc_agg_k(src_r, dst_r, h0, h1)
    return _tc2(h0, h1, ah0, ah1, deg0, deg1, V2, b2r)
